# baseline (speedup 1.0000x reference)
import functools

import numpy as np
import jax
import jax.numpy as jnp
from jax import lax
from jax.experimental import pallas as pl
from jax.experimental.pallas import tpu as pltpu
from jax.experimental.pallas import tpu_sc as plsc

_VOCAB = 1000000
_B = 16384
_ND = 13
_NS = 26
_ED = 128
_NF = _NS + 1

_r, _c = np.triu_indices(_NF)
_PAIR = np.zeros((_NF, _NF), dtype=np.int32)
_PAIR[_r, _c] = np.arange(_r.size, dtype=np.int32)
_PAIR[_c, _r] = _PAIR[_r, _c]
_NP32 = 32
_INV_EE = np.zeros((_NP32, _NP32), dtype=np.int32)
_SCALE_EE = np.zeros((_NP32, _NP32), dtype=np.float32)
for _i in range(_NS):
    for _j in range(_NS):
        _INV_EE[_i, _j] = _PAIR[_i + 1, _j + 1]
        _SCALE_EE[_i, _j] = 1.0 if _i == _j else 0.5
_INV_EE = _INV_EE.reshape(-1)
_SCALE_EE = _SCALE_EE.reshape(-1)
_INV_X = np.zeros((_NP32,), dtype=np.int32)
_SCALE_X = np.zeros((_NP32,), dtype=np.float32)
for _i in range(_NS):
    _INV_X[_i] = _PAIR[0, _i + 1]
    _SCALE_X[_i] = 1.0
_DD = int(_PAIR[0, 0])

_NC = 2
_NSUB = 16
_NW = _NC * _NSUB
_CH = 128
_TOT = _B * _NS
_BPW = _TOT // _NW
_NCHUNK = _BPW // _CH


_GRP = 2
_GRPC = _GRP * _CH


def _sc_gather(table, idx):
    tot = idx.shape[0]
    bpw = tot // _NW
    nchunk = bpw // _CH
    npair = nchunk // (2 * _GRP)
    assert npair * 2 * _GRP == nchunk and nchunk * _CH == bpw
    max_delta = max((w * nchunk) % 8 for w in range(_NW))
    ncopy = nchunk + max_delta
    idx2 = idx.reshape(tot // _CH, _CH)
    mesh = plsc.VectorSubcoreMesh(core_axis_name="c", subcore_axis_name="s")

    @functools.partial(
        pl.kernel,
        out_type=jax.ShapeDtypeStruct((tot, _ED), jnp.float32),
        mesh=mesh,
        scratch_types=[
            pltpu.VMEM((ncopy, _CH), jnp.int32),
            pltpu.VMEM((2, _GRPC, _ED), jnp.float32),
            pltpu.SemaphoreType.DMA,
            pltpu.SemaphoreType.DMA,
            pltpu.SemaphoreType.DMA,
            pltpu.SemaphoreType.DMA,
        ],
    )
    def k(idx_hbm, table_hbm, out_hbm, idx_v, rows_v, gs0, gs1, os0, os1):
        wid = lax.axis_index("s") * _NC + lax.axis_index("c")
        start = wid * nchunk
        astart = pl.multiple_of(start // 8 * 8, 8)
        delta = start - astart
        pltpu.sync_copy(idx_hbm.at[pl.ds(astart, ncopy)], idx_v)
        gsems = (gs0, gs1)
        osems = (os0, os1)

        def out_cp(buf, grp):
            dst = out_hbm.at[pl.ds(wid * bpw + grp * _GRPC, _GRPC)]
            return pltpu.make_async_copy(rows_v.at[buf], dst, osems[buf])

        def body(it, carry):
            gathers = []
            for buf in range(2):
                grp = 2 * it + buf

                @pl.when(it > 0)
                def _drain():
                    out_cp(buf, 0).wait()

                for c in range(_GRP):
                    j = grp * _GRP + c
                    cp = pltpu.make_async_copy(
                        table_hbm.at[idx_v.at[delta + j]],
                        rows_v.at[buf].at[pl.ds(c * _CH, _CH)],
                        gsems[buf],
                    )
                    cp.start()
                    gathers.append(cp)
            for buf in range(2):
                for c in range(_GRP):
                    gathers[buf * _GRP + c].wait()
                out_cp(buf, 2 * it + buf).start()
            return carry

        lax.fori_loop(0, npair, body, 0)
        for buf in range(2):
            out_cp(buf, 0).wait()

    return k(idx2, table)


_BB = 512


def _tc_body(dense_ref, emb_lo, emb_hi, wb0, bb0, wb1, bb1, wb2, bb2, w0a, w2,
             w2x, w2dd, bt0, wt1, bt1, wt2, bt2, wt3, bt3, wt4, bt4, out_ref):
    f32 = jnp.float32
    d = dense_ref[...]
    h = jnp.maximum(jnp.dot(d, wb0[...], preferred_element_type=f32) + bb0[...], 0.0)
    h = jnp.maximum(jnp.dot(h, wb1[...], preferred_element_type=f32) + bb1[...], 0.0)
    ed = jnp.maximum(jnp.dot(h, wb2[...], preferred_element_type=f32) + bb2[...], 0.0)
    hb = _BB // 2
    for half, eref in ((0, emb_lo), (1, emb_hi)):
        edh = ed[half * hb:(half + 1) * hb]
        cc = eref[...].reshape(hb, _NP32, _ED)
        g = lax.dot_general(cc, cc, (((2,), (2,)), ((0,), (0,))),
                            preferred_element_type=f32)
        gx = lax.dot_general(edh, cc, (((1,), (2,)), ((0,), (0,))),
                             preferred_element_type=f32)
        ed2 = jnp.sum(edh * edh, axis=1, keepdims=True)
        gf = g.reshape(hb, _NP32 * _NP32)
        t = (jnp.dot(edh, w0a[...], preferred_element_type=f32)
             + jnp.dot(gf, w2[...], preferred_element_type=f32)
             + jnp.dot(gx, w2x[...], preferred_element_type=f32)
             + ed2 * w2dd[...] + bt0[...])
        t = jnp.maximum(t, 0.0)
        t = jnp.maximum(jnp.dot(t, wt1[...], preferred_element_type=f32) + bt1[...], 0.0)
        t = jnp.maximum(jnp.dot(t, wt2[...], preferred_element_type=f32) + bt2[...], 0.0)
        t = jnp.maximum(jnp.dot(t, wt3[...], preferred_element_type=f32) + bt3[...], 0.0)
        out_ref[half * hb:(half + 1) * hb, :] = (
            jnp.dot(t, wt4[...], preferred_element_type=f32) + bt4[...])


def _full(a):
    return pl.BlockSpec(a.shape, lambda i: (0,) * a.ndim)


def _tc_fused(dense, emb2d, weights):
    nblk = dense.shape[0] // _BB
    hrows = _BB * _NP32 // 2
    in_specs = [
        pl.BlockSpec((_BB, _ND), lambda i: (i, 0)),
        pl.BlockSpec((hrows, _ED), lambda i: (2 * i, 0)),
        pl.BlockSpec((hrows, _ED), lambda i: (2 * i + 1, 0)),
    ] + [_full(w) for w in weights]
    return pl.pallas_call(
        _tc_body,
        grid=(nblk,),
        in_specs=in_specs,
        out_specs=pl.BlockSpec((_BB, 1), lambda i: (i, 0)),
        out_shape=jax.ShapeDtypeStruct((dense.shape[0], 1), jnp.float32),
    )(dense, emb2d, emb2d, *weights)


def kernel(x, emb_table, Wb0, bb0, Wb1, bb1, Wb2, bb2, Wt0, bt0, Wt1, bt1,
           Wt2, bt2, Wt3, bt3, Wt4, bt4):
    dense = x[:, :_ND]
    sp = x[:, _ND:].astype(jnp.int32) % _VOCAB
    idx32 = jnp.concatenate(
        [sp, jnp.zeros((_B, _NP32 - _NS), jnp.int32)], axis=1
    ).reshape(-1)
    wt0i = Wt0[_ED:]
    w0a = Wt0[:_ED]
    w2 = wt0i[_INV_EE] * _SCALE_EE[:, None]
    w2x = wt0i[_INV_X] * _SCALE_X[:, None]
    w2dd = wt0i[_DD].reshape(1, -1)
    weights = (
        Wb0, bb0.reshape(1, -1), Wb1, bb1.reshape(1, -1), Wb2,
        bb2.reshape(1, -1), w0a, w2, w2x, w2dd, bt0.reshape(1, -1), Wt1,
        bt1.reshape(1, -1), Wt2, bt2.reshape(1, -1), Wt3,
        bt3.reshape(1, -1), Wt4, bt4.reshape(1, -1),
    )
    h = _B // 2
    hr = h * _NP32
    outs = []
    for s in range(2):
        emb2d = _sc_gather(emb_table, idx32[s * hr:(s + 1) * hr])
        outs.append(_tc_fused(dense[s * h:(s + 1) * h], emb2d, weights))
    return jnp.concatenate(outs, axis=0)


# speedup vs baseline: 11.3691x; 11.3691x over previous
import functools

import numpy as np
import jax
import jax.numpy as jnp
from jax import lax
from jax.experimental import pallas as pl
from jax.experimental.pallas import tpu as pltpu
from jax.experimental.pallas import tpu_sc as plsc

_VOCAB = 1000000
_B = 16384
_ND = 13
_NS = 26
_ED = 128
_NF = _NS + 1

_r, _c = np.triu_indices(_NF)
_PAIR = np.zeros((_NF, _NF), dtype=np.int32)
_PAIR[_r, _c] = np.arange(_r.size, dtype=np.int32)
_PAIR[_c, _r] = _PAIR[_r, _c]
_PERM = np.concatenate([np.arange(1, _NF), [0]])
_PAIRP = _PAIR[np.ix_(_PERM, _PERM)]
_INV = _PAIRP.reshape(-1)
_SCALE = np.where(np.eye(_NF, dtype=bool), 1.0, 0.5).reshape(-1).astype(np.float32)

_NC = 2
_NSUB = 16
_NW = _NC * _NSUB
_CH = 128
_GRP = 2
_GRPC = _GRP * _CH


def _sc_gather(table, idx):
    tot = idx.shape[0]
    bpw = tot // _NW
    nchunk = bpw // _CH
    grp = _GRP if nchunk % (2 * _GRP) == 0 else 1
    grpc = grp * _CH
    npair = nchunk // (2 * grp)
    assert npair * 2 * grp == nchunk and nchunk * _CH == bpw
    max_delta = max((w * nchunk) % 8 for w in range(_NW))
    ncopy = nchunk + max_delta
    idx2 = idx.reshape(tot // _CH, _CH)
    mesh = plsc.VectorSubcoreMesh(core_axis_name="c", subcore_axis_name="s")

    @functools.partial(
        pl.kernel,
        out_type=jax.ShapeDtypeStruct((tot, _ED), jnp.float32),
        mesh=mesh,
        scratch_types=[
            pltpu.VMEM((ncopy, _CH), jnp.int32),
            pltpu.VMEM((2, grpc, _ED), jnp.float32),
            pltpu.SemaphoreType.DMA,
            pltpu.SemaphoreType.DMA,
            pltpu.SemaphoreType.DMA,
            pltpu.SemaphoreType.DMA,
        ],
    )
    def k(idx_hbm, table_hbm, out_hbm, idx_v, rows_v, gs0, gs1, os0, os1):
        wid = lax.axis_index("s") * _NC + lax.axis_index("c")
        start = wid * nchunk
        astart = pl.multiple_of(start // 8 * 8, 8)
        delta = start - astart
        pltpu.sync_copy(idx_hbm.at[pl.ds(astart, ncopy)], idx_v)
        gsems = (gs0, gs1)
        osems = (os0, os1)

        def out_cp(buf, g):
            dst = out_hbm.at[pl.ds(wid * bpw + g * grpc, grpc)]
            return pltpu.make_async_copy(rows_v.at[buf], dst, osems[buf])

        def body(it, carry):
            gathers = []
            for buf in range(2):
                g = 2 * it + buf

                @pl.when(it > 0)
                def _drain():
                    out_cp(buf, 0).wait()

                for c in range(grp):
                    j = g * grp + c
                    cp = pltpu.make_async_copy(
                        table_hbm.at[idx_v.at[delta + j]],
                        rows_v.at[buf].at[pl.ds(c * _CH, _CH)],
                        gsems[buf],
                    )
                    cp.start()
                    gathers.append(cp)
            for buf in range(2):
                for c in range(grp):
                    gathers[buf * grp + c].wait()
                out_cp(buf, 2 * it + buf).start()
            return carry

        lax.fori_loop(0, npair, body, 0)
        for buf in range(2):
            out_cp(buf, 0).wait()

    return k(idx2, table)


_BB = 512


def _tc_body(dense_ref, emb_ref, wb0, bb0, wb1, bb1, wb2, bb2, w0a, w2,
             bt0, wt1, bt1, wt2, bt2, wt3, bt3, wt4, bt4, out_ref):
    f32 = jnp.float32
    d = dense_ref[...]
    h = jnp.maximum(jnp.dot(d, wb0[...], preferred_element_type=f32) + bb0[...], 0.0)
    h = jnp.maximum(jnp.dot(h, wb1[...], preferred_element_type=f32) + bb1[...], 0.0)
    ed = jnp.maximum(jnp.dot(h, wb2[...], preferred_element_type=f32) + bb2[...], 0.0)
    emb3 = emb_ref[...].reshape(_BB, _NS, _ED)
    cc = jnp.concatenate([emb3, ed[:, None, :]], axis=1)
    g = lax.dot_general(cc, cc, (((2,), (2,)), ((0,), (0,))),
                        preferred_element_type=f32)
    gf = g.reshape(_BB, _NF * _NF)
    t = (jnp.dot(ed, w0a[...], preferred_element_type=f32)
         + jnp.dot(gf, w2[...], preferred_element_type=f32) + bt0[...])
    t = jnp.maximum(t, 0.0)
    t = jnp.maximum(jnp.dot(t, wt1[...], preferred_element_type=f32) + bt1[...], 0.0)
    t = jnp.maximum(jnp.dot(t, wt2[...], preferred_element_type=f32) + bt2[...], 0.0)
    t = jnp.maximum(jnp.dot(t, wt3[...], preferred_element_type=f32) + bt3[...], 0.0)
    out_ref[...] = jnp.dot(t, wt4[...], preferred_element_type=f32) + bt4[...]


def _full(a):
    return pl.BlockSpec(a.shape, lambda i: (0,) * a.ndim)


def _tc_fused(dense, emb2d, weights):
    nblk = dense.shape[0] // _BB
    in_specs = [
        pl.BlockSpec((_BB, _ND), lambda i: (i, 0)),
        pl.BlockSpec((_BB * _NS, _ED), lambda i: (i, 0)),
    ] + [_full(w) for w in weights]
    return pl.pallas_call(
        _tc_body,
        grid=(nblk,),
        in_specs=in_specs,
        out_specs=pl.BlockSpec((_BB, 1), lambda i: (i, 0)),
        out_shape=jax.ShapeDtypeStruct((dense.shape[0], 1), jnp.float32),
    )(dense, emb2d, *weights)


_NSLICE = 4


def kernel(x, emb_table, Wb0, bb0, Wb1, bb1, Wb2, bb2, Wt0, bt0, Wt1, bt1,
           Wt2, bt2, Wt3, bt3, Wt4, bt4):
    dense = x[:, :_ND]
    idx = (x[:, _ND:].astype(jnp.int32) % _VOCAB).reshape(-1)
    w0a = Wt0[:_ED]
    w2 = Wt0[_ED:][_INV] * _SCALE[:, None]
    weights = (
        Wb0, bb0.reshape(1, -1), Wb1, bb1.reshape(1, -1), Wb2,
        bb2.reshape(1, -1), w0a, w2, bt0.reshape(1, -1), Wt1,
        bt1.reshape(1, -1), Wt2, bt2.reshape(1, -1), Wt3,
        bt3.reshape(1, -1), Wt4, bt4.reshape(1, -1),
    )
    h = _B // _NSLICE
    hr = h * _NS
    outs = []
    for s in range(_NSLICE):
        emb2d = _sc_gather(emb_table, idx[s * hr:(s + 1) * hr])
        outs.append(_tc_fused(dense[s * h:(s + 1) * h], emb2d, weights))
    return jnp.concatenate(outs, axis=0)
